# baseline (speedup 1.0000x reference)
import functools

import jax
import jax.numpy as jnp
from jax import lax
from jax.experimental import pallas as pl
from jax.experimental.pallas import tpu as pltpu
from jax.experimental.pallas import tpu_sc as plsc

_DIM = 64
_LANES = 128
_NTOK = 1000000
_FULL_COLS = _NTOK // _LANES
_SCR_ROWS = 500032
_BATCH = 4096
_HIST = 200


def _workers():
    info = plsc.get_sparse_core_info()
    return info.num_cores, info.num_subcores


@functools.cache
def _transpose_call():
    num_cores, num_subcores = _workers()
    num_workers = num_cores * num_subcores
    n_super = _FULL_COLS // 2
    per_worker = 2 * (-(-n_super // (2 * num_workers)))
    n_pairs = per_worker // 2
    mesh = plsc.VectorSubcoreMesh(core_axis_name="c", subcore_axis_name="s")

    @functools.partial(
        pl.kernel,
        mesh=mesh,
        compiler_params=pltpu.CompilerParams(use_tc_tiling_on_sc=True, needs_layout_passes=False),
        out_type=jax.ShapeDtypeStruct((_SCR_ROWS, _LANES), jnp.float32),
        scratch_types=[
            pltpu.VMEM((_DIM, 2 * _LANES), jnp.float32),
            pltpu.VMEM((_DIM, 2 * _LANES), jnp.float32),
            pltpu.VMEM((_LANES, _LANES), jnp.float32),
            pltpu.VMEM((_LANES, _LANES), jnp.float32),
            pltpu.SemaphoreType.DMA,
            pltpu.SemaphoreType.DMA,
            pltpu.SemaphoreType.DMA,
            pltpu.SemaphoreType.DMA,
        ],
    )
    def tr(wt_hbm, tail_hbm, scr_hbm, in0, in1, ro0, ro1, si0, si1, so0, so1):
        wid = lax.axis_index("s") * num_cores + lax.axis_index("c")
        start = wid * per_worker
        ins = (in0, in1)
        rows = (ro0, ro1)
        sis = (si0, si1)
        sos = (so0, so1)
        lane16 = lax.iota(jnp.int32, 16)

        def stage(c, b, begin):
            src = wt_hbm.at[
                :, pl.ds(pl.multiple_of(c * 2 * _LANES, 2 * _LANES), 2 * _LANES)
            ]
            if begin:
                pltpu.async_copy(src, ins[b], sis[b])
            else:
                pltpu.make_async_copy(src, ins[b], sis[b]).wait()

        def flush(c, b, begin):
            dst = scr_hbm.at[
                pl.ds(pl.multiple_of(c * _LANES, _LANES), _LANES), :
            ]
            if begin:
                pltpu.async_copy(rows[b], dst, sos[b])
            else:
                pltpu.make_async_copy(rows[b], dst, sos[b]).wait()

        def transpose(b, n_jblk):
            def body(s, carry):
                wrap = (lane16 + s) & 15
                for kd in range(_DIM // 16):
                    d0 = 16 * kd
                    dvec = lane16 + d0
                    for kj in range(n_jblk):
                        jv = wrap + 16 * kj
                        v = plsc.load_gather(ins[b], [dvec, jv])
                        plsc.store_scatter(
                            rows[b],
                            [lax.shift_right_logical(jv, 1), ((jv & 1) << 6) + dvec],
                            v,
                        )
                return carry

            lax.fori_loop(0, 16, body, 0)

        def col_section(c, b):
            @pl.when(c < n_super)
            def _():
                stage(c, b, False)

                @pl.when(c - 2 >= start)
                def _():
                    flush(c - 2, b, False)

                transpose(b, 2 * _LANES // 16)
                flush(c, b, True)

            nxt = c + 2

            @pl.when(nxt - start < per_worker)
            def _():
                @pl.when(nxt < n_super)
                def _():
                    stage(nxt, b, True)

        @pl.when(start < n_super)
        def _():
            stage(start, 0, True)

        @pl.when(start + 1 < n_super)
        def _():
            stage(start + 1, 1, True)

        def pair_body(p, carry):
            col_section(start + 2 * p, 0)
            col_section(start + 2 * p + 1, 1)
            return carry

        lax.fori_loop(0, n_pairs, pair_body, 0)

        flush(start, 0, False)
        flush(start + 1, 1, False)

        @pl.when(wid == num_workers - 1)
        def _():
            pltpu.async_copy(tail_hbm, in0.at[:, pl.ds(0, _LANES)], si0).wait()
            transpose(0, _LANES // 16)
            dst = scr_hbm.at[pl.ds((_NTOK - _LANES) // 2, _DIM), :]
            pltpu.async_copy(rows[0].at[pl.ds(0, _DIM), :], dst, so0).wait()

    return tr


@functools.cache
def _gather_call():
    num_cores, num_subcores = _workers()
    num_workers = num_cores * num_subcores
    n_hblk = _HIST // 8
    n_bblk = _BATCH // _LANES
    tiles_per_worker = n_hblk * n_bblk // num_workers
    mesh = plsc.VectorSubcoreMesh(core_axis_name="c", subcore_axis_name="s")

    @functools.partial(
        pl.kernel,
        mesh=mesh,
        compiler_params=pltpu.CompilerParams(use_tc_tiling_on_sc=True, needs_layout_passes=False),
        out_type=jax.ShapeDtypeStruct((_HIST, _DIM, _BATCH), jnp.float32),
        scratch_types=[
            pltpu.VMEM((8, _LANES), jnp.int32),
            pltpu.VMEM((_LANES,), jnp.int32),
            pltpu.VMEM((_LANES,), jnp.int32),
            pltpu.VMEM((_LANES,), jnp.int32),
            pltpu.VMEM((_LANES,), jnp.int32),
            pltpu.VMEM((_LANES,), jnp.int32),
            pltpu.VMEM((_LANES,), jnp.int32),
            pltpu.VMEM((_LANES, _LANES), jnp.float32),
            pltpu.VMEM((_LANES, _LANES), jnp.float32),
            pltpu.VMEM((_LANES, _LANES), jnp.float32),
            pltpu.VMEM((_DIM, _LANES), jnp.float32),
            pltpu.VMEM((_DIM, _LANES), jnp.float32),
            pltpu.SemaphoreType.DMA,
            pltpu.SemaphoreType.DMA,
            pltpu.SemaphoreType.DMA,
            pltpu.SemaphoreType.DMA,
            pltpu.SemaphoreType.DMA,
        ],
    )
    def ga(scr_hbm, xt_hbm, out_hbm, idx_t, pi0, pi1, pi2, pa0, pa1, pa2,
           gb0, gb1, gb2, os0, os1, sg0, sg1, sg2, so0, so1):
        wid = lax.axis_index("s") * num_cores + lax.axis_index("c")
        pis = (pi0, pi1, pi2)
        pas = (pa0, pa1, pa2)
        gbs = (gb0, gb1, gb2)
        oss = (os0, os1)
        sgs = (sg0, sg1, sg2)
        sos = (so0, so1)
        lane16 = lax.iota(jnp.int32, 16)

        def prep(hh, b):
            for m in range(8):
                v = plsc.load_gather(
                    idx_t, [jnp.broadcast_to(hh, (16,)), lane16 + 16 * m]
                )
                pis[b][pl.ds(16 * m, 16)] = lax.shift_right_logical(v, 1)
                pas[b][pl.ds(16 * m, 16)] = (v & 1) * _DIM

        def gather(b, begin):
            src = scr_hbm.at[pis[b]]
            if begin:
                pltpu.async_copy(src, gbs[b], sgs[b])
            else:
                pltpu.make_async_copy(src, gbs[b], sgs[b]).wait()

        def flush(h, c, b, begin):
            dst = out_hbm.at[h, :, pl.ds(pl.multiple_of(c * _LANES, _LANES), _LANES)]
            if begin:
                pltpu.async_copy(oss[b], dst, sos[b])
            else:
                pltpu.make_async_copy(oss[b], dst, sos[b]).wait()

        def transpose(b, ob):
            pars = [pas[b][pl.ds(16 * m, 16)] for m in range(8)]
            jrows = [lane16 + 16 * m for m in range(8)]

            def body(s, carry):
                wrap = (lane16 + s) & 15
                for kd in range(_DIM // 16):
                    d0 = 16 * kd
                    dvec = wrap + d0
                    for m in range(8):
                        v = plsc.load_gather(gbs[b], [jrows[m], pars[m] + dvec])
                        plsc.store_scatter(oss[ob], [dvec, jrows[m]], v)
                return carry

            lax.fori_loop(0, 16, body, 0)

        def tile_body(tt, carry):
            t = wid * tiles_per_worker + tt
            h0 = t // n_bblk
            c = t % n_bblk
            pltpu.sync_copy(
                xt_hbm.at[
                    pl.ds(pl.multiple_of(h0 * 8, 8), 8),
                    pl.ds(pl.multiple_of(c * _LANES, _LANES), _LANES),
                ],
                idx_t,
            )
            prep(0, 0)
            gather(0, True)
            prep(1, 1)
            gather(1, True)
            for hh in range(8):
                b = hh % 3
                if hh < 6:
                    nb = (hh + 2) % 3
                    prep(hh + 2, nb)
                    gather(nb, True)
                gather(b, False)
                ob = hh % 2
                if hh >= 2:
                    flush(h0 * 8 + hh - 2, c, ob, False)
                transpose(b, ob)
                flush(h0 * 8 + hh, c, ob, True)
            for hh in (6, 7):
                flush(h0 * 8 + hh, c, hh % 2, False)
            return carry

        lax.fori_loop(0, tiles_per_worker, tile_body, 0)

    return ga


def kernel(x, emb_weight):
    wt = emb_weight.T
    tail = emb_weight[_NTOK - _LANES :, :].T
    xt = x.astype(jnp.int32).T
    scr = _transpose_call()(wt, tail)
    out_t = _gather_call()(scr, xt)
    return out_t.transpose(2, 0, 1)


# speedup vs baseline: 1.9797x; 1.9797x over previous
import functools

import jax
import jax.numpy as jnp
from jax import lax
from jax.experimental import pallas as pl
from jax.experimental.pallas import tpu as pltpu
from jax.experimental.pallas import tpu_sc as plsc

_DIM = 64
_LANES = 128
_NTOK = 1000000
_FULL_COLS = _NTOK // _LANES
_SCR_ROWS = 500032
_BATCH = 4096
_HIST = 200


def _workers():
    info = plsc.get_sparse_core_info()
    return info.num_cores, info.num_subcores


@functools.cache
def _transpose_call():
    num_cores, num_subcores = _workers()
    num_workers = num_cores * num_subcores
    n_super = _FULL_COLS // 2
    per_worker = 2 * (-(-n_super // (2 * num_workers)))
    n_pairs = per_worker // 2
    mesh = plsc.VectorSubcoreMesh(core_axis_name="c", subcore_axis_name="s")

    @functools.partial(
        pl.kernel,
        mesh=mesh,
        compiler_params=pltpu.CompilerParams(use_tc_tiling_on_sc=True, needs_layout_passes=False),
        out_type=jax.ShapeDtypeStruct((_SCR_ROWS, _LANES), jnp.float32),
        scratch_types=[
            pltpu.VMEM((_DIM, 2 * _LANES), jnp.float32),
            pltpu.VMEM((_DIM, 2 * _LANES), jnp.float32),
            pltpu.VMEM((_LANES, _LANES), jnp.float32),
            pltpu.VMEM((_LANES, _LANES), jnp.float32),
            pltpu.SemaphoreType.DMA,
            pltpu.SemaphoreType.DMA,
            pltpu.SemaphoreType.DMA,
            pltpu.SemaphoreType.DMA,
        ],
    )
    def tr(wt_hbm, tail_hbm, scr_hbm, in0, in1, ro0, ro1, si0, si1, so0, so1):
        wid = lax.axis_index("s") * num_cores + lax.axis_index("c")
        start = wid * per_worker
        ins = (in0, in1)
        rows = (ro0, ro1)
        sis = (si0, si1)
        sos = (so0, so1)
        lane16 = lax.iota(jnp.int32, 16)

        def stage(c, b, begin):
            src = wt_hbm.at[
                :, pl.ds(pl.multiple_of(c * 2 * _LANES, 2 * _LANES), 2 * _LANES)
            ]
            if begin:
                pltpu.async_copy(src, ins[b], sis[b])
            else:
                pltpu.make_async_copy(src, ins[b], sis[b]).wait()

        def flush(c, b, begin):
            dst = scr_hbm.at[
                pl.ds(pl.multiple_of(c * _LANES, _LANES), _LANES), :
            ]
            if begin:
                pltpu.async_copy(rows[b], dst, sos[b])
            else:
                pltpu.make_async_copy(rows[b], dst, sos[b]).wait()

        def transpose(b, n_jblk):
            def body(s, carry):
                wrap = (lane16 + s) & 15
                for kd in range(_DIM // 16):
                    d0 = 16 * kd
                    dvec = lane16 + d0
                    vs = [
                        plsc.load_gather(ins[b], [dvec, wrap + 16 * kj])
                        for kj in range(n_jblk)
                    ]
                    for kj in range(n_jblk):
                        jv = wrap + 16 * kj
                        plsc.store_scatter(
                            rows[b],
                            [lax.shift_right_logical(jv, 1), ((jv & 1) << 6) + dvec],
                            vs[kj],
                        )
                return carry

            lax.fori_loop(0, 16, body, 0)

        def col_section(c, b):
            @pl.when(c < n_super)
            def _():
                stage(c, b, False)

                @pl.when(c - 2 >= start)
                def _():
                    flush(c - 2, b, False)

                transpose(b, 2 * _LANES // 16)
                flush(c, b, True)

            nxt = c + 2

            @pl.when(nxt - start < per_worker)
            def _():
                @pl.when(nxt < n_super)
                def _():
                    stage(nxt, b, True)

        @pl.when(start < n_super)
        def _():
            stage(start, 0, True)

        @pl.when(start + 1 < n_super)
        def _():
            stage(start + 1, 1, True)

        def pair_body(p, carry):
            col_section(start + 2 * p, 0)
            col_section(start + 2 * p + 1, 1)
            return carry

        lax.fori_loop(0, n_pairs, pair_body, 0)

        flush(start, 0, False)
        flush(start + 1, 1, False)

        @pl.when(wid == num_workers - 1)
        def _():
            pltpu.async_copy(tail_hbm, in0.at[:, pl.ds(0, _LANES)], si0).wait()
            transpose(0, _LANES // 16)
            dst = scr_hbm.at[pl.ds((_NTOK - _LANES) // 2, _DIM), :]
            pltpu.async_copy(rows[0].at[pl.ds(0, _DIM), :], dst, so0).wait()

    return tr


@functools.cache
def _gather_call():
    num_cores, num_subcores = _workers()
    num_workers = num_cores * num_subcores
    n_hblk = _HIST // 8
    n_bblk = _BATCH // _LANES
    tiles_per_worker = n_hblk * n_bblk // num_workers
    mesh = plsc.VectorSubcoreMesh(core_axis_name="c", subcore_axis_name="s")

    @functools.partial(
        pl.kernel,
        mesh=mesh,
        compiler_params=pltpu.CompilerParams(use_tc_tiling_on_sc=True, needs_layout_passes=False),
        out_type=jax.ShapeDtypeStruct((_HIST, _DIM, _BATCH), jnp.float32),
        scratch_types=[
            pltpu.VMEM((8, _LANES), jnp.int32),
            pltpu.VMEM((_LANES,), jnp.int32),
            pltpu.VMEM((_LANES,), jnp.int32),
            pltpu.VMEM((_LANES,), jnp.int32),
            pltpu.VMEM((_LANES,), jnp.int32),
            pltpu.VMEM((_LANES,), jnp.int32),
            pltpu.VMEM((_LANES,), jnp.int32),
            pltpu.VMEM((_LANES, _LANES), jnp.float32),
            pltpu.VMEM((_LANES, _LANES), jnp.float32),
            pltpu.VMEM((_LANES, _LANES), jnp.float32),
            pltpu.VMEM((_DIM, _LANES), jnp.float32),
            pltpu.VMEM((_DIM, _LANES), jnp.float32),
            pltpu.SemaphoreType.DMA,
            pltpu.SemaphoreType.DMA,
            pltpu.SemaphoreType.DMA,
            pltpu.SemaphoreType.DMA,
            pltpu.SemaphoreType.DMA,
        ],
    )
    def ga(scr_hbm, xt_hbm, out_hbm, idx_t, pi0, pi1, pi2, pa0, pa1, pa2,
           gb0, gb1, gb2, os0, os1, sg0, sg1, sg2, so0, so1):
        wid = lax.axis_index("s") * num_cores + lax.axis_index("c")
        pis = (pi0, pi1, pi2)
        pas = (pa0, pa1, pa2)
        gbs = (gb0, gb1, gb2)
        oss = (os0, os1)
        sgs = (sg0, sg1, sg2)
        sos = (so0, so1)
        lane16 = lax.iota(jnp.int32, 16)

        def prep(hh, b):
            for m in range(8):
                v = plsc.load_gather(
                    idx_t, [jnp.broadcast_to(hh, (16,)), lane16 + 16 * m]
                )
                pis[b][pl.ds(16 * m, 16)] = lax.shift_right_logical(v, 1)
                pas[b][pl.ds(16 * m, 16)] = (v & 1) * _DIM

        def gather(b, begin):
            src = scr_hbm.at[pis[b]]
            if begin:
                pltpu.async_copy(src, gbs[b], sgs[b])
            else:
                pltpu.make_async_copy(src, gbs[b], sgs[b]).wait()

        def flush(h, c, b, begin):
            dst = out_hbm.at[h, :, pl.ds(pl.multiple_of(c * _LANES, _LANES), _LANES)]
            if begin:
                pltpu.async_copy(oss[b], dst, sos[b])
            else:
                pltpu.make_async_copy(oss[b], dst, sos[b]).wait()

        def transpose(b, ob):
            pars = [pas[b][pl.ds(16 * m, 16)] for m in range(8)]
            jrows = [lane16 + 16 * m for m in range(8)]

            def body(s, carry):
                wrap = (lane16 + s) & 15
                for kd in range(_DIM // 16):
                    d0 = 16 * kd
                    dvec = wrap + d0
                    vs = [
                        plsc.load_gather(gbs[b], [jrows[m], pars[m] + dvec])
                        for m in range(8)
                    ]
                    for m in range(8):
                        plsc.store_scatter(oss[ob], [dvec, jrows[m]], vs[m])
                return carry

            lax.fori_loop(0, 16, body, 0)

        def tile_body(tt, carry):
            t = wid * tiles_per_worker + tt
            h0 = t // n_bblk
            c = t % n_bblk
            pltpu.sync_copy(
                xt_hbm.at[
                    pl.ds(pl.multiple_of(h0 * 8, 8), 8),
                    pl.ds(pl.multiple_of(c * _LANES, _LANES), _LANES),
                ],
                idx_t,
            )
            prep(0, 0)
            gather(0, True)
            prep(1, 1)
            gather(1, True)
            for hh in range(8):
                b = hh % 3
                if hh < 6:
                    nb = (hh + 2) % 3
                    prep(hh + 2, nb)
                    gather(nb, True)
                gather(b, False)
                ob = hh % 2
                if hh >= 2:
                    flush(h0 * 8 + hh - 2, c, ob, False)
                transpose(b, ob)
                flush(h0 * 8 + hh, c, ob, True)
            for hh in (6, 7):
                flush(h0 * 8 + hh, c, hh % 2, False)
            return carry

        lax.fori_loop(0, tiles_per_worker, tile_body, 0)

    return ga


def kernel(x, emb_weight):
    wt = emb_weight.T
    tail = emb_weight[_NTOK - _LANES :, :].T
    xt = x.astype(jnp.int32).T
    scr = _transpose_call()(wt, tail)
    out_t = _gather_call()(scr, xt)
    return out_t.transpose(2, 0, 1)
